# baseline (speedup 1.0000x reference)
import functools

import jax
import jax.numpy as jnp
from jax import lax
from jax.experimental import pallas as pl
from jax.experimental.pallas import tpu as pltpu
from jax.experimental.pallas import tpu_sc as plsc

NEG_SLOPE = 0.2
NC, NS, L = 2, 16, 16
NW = NC * NS
B = 96
DG = 8


def _tc_pre(x, W, att_src, att_dst, npad):
    n, d = x.shape

    def body(x_ref, w_ref, s_ref, t_ref, h_ref, as_ref, ad_ref, c_ref):
        h = jnp.dot(x_ref[...], w_ref[...], preferred_element_type=jnp.float32)
        h_ref[...] = h
        a_s = jnp.sum(h * s_ref[...][None, :], axis=1)
        a_d = jnp.sum(h * t_ref[...][None, :], axis=1)
        pad = jnp.zeros((npad - n,), jnp.float32)
        as_ref[...] = jnp.concatenate([a_s, pad])
        ad_ref[...] = jnp.concatenate([a_d, pad])
        c = jnp.maximum(jnp.max(a_s) + jnp.max(a_d), 0.0)
        c_ref[...] = jnp.full((L,), c, jnp.float32)

    return pl.pallas_call(
        body,
        out_shape=(
            jax.ShapeDtypeStruct((n, d), jnp.float32),
            jax.ShapeDtypeStruct((npad,), jnp.float32),
            jax.ShapeDtypeStruct((npad,), jnp.float32),
            jax.ShapeDtypeStruct((L,), jnp.float32),
        ),
    )(x, W, att_src, att_dst)


def _tc_post(acc, den, bias, n):
    def body(a_ref, d_ref, b_ref, o_ref):
        a = a_ref[0] + a_ref[1]
        den_sum = d_ref[0] + d_ref[1] + 1e-16
        o_ref[...] = a[:n] / den_sum[:n, None] + b_ref[...][None, :]

    return pl.pallas_call(
        body,
        out_shape=jax.ShapeDtypeStruct((n, acc.shape[2]), jnp.float32),
    )(acc, den, bias)


def _sc_edge_kernel(npad, epad, d):
    epb = epad // NW
    nch = epb // B
    rpt = npad // NS
    assert nch % 6 == 0 and rpt % 8 == 0

    def body(src_hbm, dst_hbm, h_hbm, asrc_hbm, adst_hbm, c_hbm, zrow_hbm,
             zvec_hbm, acc_hbm, den_hbm,
             asrc_v, adst_v, c_v, sidx0, sidx1, sidx2, didx0, didx1, didx2,
             p0, p1, rows0, rows1, acc_sh, den_sh,
             gs0, gs1, ss0, ss1, is0, is1, is2):
        cid = lax.axis_index("c")
        sid = lax.axis_index("s")
        wid = cid * NS + sid
        sidx = (sidx0, sidx1, sidx2)
        didx = (didx0, didx1, didx2)
        pb = (p0, p1)
        rows = (rows0, rows1)
        gsem = (gs0, gs1)
        ssem = (ss0, ss1)
        isem = (is0, is1, is2)
        base = wid * epb

        def idx_load(g, s3, sync):
            fn = pltpu.sync_copy if sync else (
                lambda s_, d_: pltpu.async_copy(s_, d_, isem[s3]))
            fn(src_hbm.at[pl.ds(base + g * B, B)], sidx[s3])
            fn(dst_hbm.at[pl.ds(base + g * B, B)], didx[s3])

        def idx_wait(g, s3):
            pltpu.make_async_copy(
                src_hbm.at[pl.ds(base + g * B, B)], sidx[s3],
                isem[s3]).wait()
            pltpu.make_async_copy(
                dst_hbm.at[pl.ds(base + g * B, B)], didx[s3],
                isem[s3]).wait()

        def gather_start(s3, s2):
            pltpu.async_copy(h_hbm.at[sidx[s3]], rows[s2], gsem[s2])

        def gather_wait(s3, s2):
            pltpu.make_async_copy(
                h_hbm.at[sidx[s3]], rows[s2], gsem[s2]).wait()

        def scatter_start(s3, s2):
            pltpu.async_copy(rows[s2], acc_sh.at[didx[s3]],
                             ssem[s2], add=True)
            pltpu.async_copy(pb[s2], den_sh.at[didx[s3]],
                             ssem[s2], add=True)

        def scatter_wait(s3, s2):
            pltpu.make_async_copy(rows[s2], acc_sh.at[didx[s3]],
                                  ssem[s2]).wait()
            pltpu.make_async_copy(pb[s2], den_sh.at[didx[s3]],
                                  ssem[s2]).wait()

        pltpu.sync_copy(zrow_hbm, acc_sh.at[pl.ds(sid * rpt, rpt)])
        pltpu.sync_copy(zvec_hbm.at[pl.ds(sid * rpt, rpt)],
                        den_sh.at[pl.ds(sid * rpt, rpt)])
        pltpu.sync_copy(asrc_hbm, asrc_v)
        pltpu.sync_copy(adst_hbm, adst_v)
        pltpu.sync_copy(c_hbm, c_v)
        idx_load(0, 0, True)
        idx_load(1, 1, True)
        gather_start(0, 0)
        gather_start(1, 1)
        idx_load(2, 2, False)
        plsc.subcore_barrier()

        @pl.loop(0, nch // 6)
        def _t(t):
            for i in range(6):
                g = t * 6 + i
                s2, s3 = i % 2, i % 3

                @pl.when(g >= 1)
                def _drain():
                    scatter_wait((i - 1) % 3, (i - 1) % 2)

                @pl.when((g >= 1) & (g + 2 < nch))
                def _pf_idx():
                    idx_load(g + 2, (i + 2) % 3, False)

                @pl.when((g + 1 >= 2) & (g + 1 < nch))
                def _pf_rows():
                    idx_wait(g + 1, (i + 1) % 3)
                    gather_start((i + 1) % 3, (i + 1) % 2)

                gather_wait(s3, s2)

                @plsc.parallel_loop(0, B, step=L, unroll=4)
                def _p(j):
                    j = pl.multiple_of(j, L)
                    sv = sidx[s3][pl.ds(j, L)]
                    dv = didx[s3][pl.ds(j, L)]
                    e = (plsc.load_gather(asrc_v, [sv])
                         + plsc.load_gather(adst_v, [dv]))
                    e = jnp.where(e > 0, e, NEG_SLOPE * e)
                    pb[s2][pl.ds(j, L)] = jnp.exp(e - c_v[...])

                @plsc.parallel_loop(0, B, unroll=8)
                def _scale(j):
                    pj = plsc.load_gather(
                        pb[s2], [jnp.full((L,), 0, jnp.int32) + j])
                    for k in range(DG):
                        rows[s2][j, pl.ds(k * L, L)] = (
                            rows[s2][j, pl.ds(k * L, L)] * pj)

                scatter_start(s3, s2)

        scatter_wait((nch - 1) % 3, (nch - 1) % 2)
        plsc.subcore_barrier()
        pltpu.sync_copy(acc_sh.at[pl.ds(sid * rpt, rpt)],
                        acc_hbm.at[cid, pl.ds(sid * rpt, rpt)])
        pltpu.sync_copy(den_sh.at[pl.ds(sid * rpt, rpt)],
                        den_hbm.at[cid, pl.ds(sid * rpt, rpt)])

    mesh = plsc.VectorSubcoreMesh(core_axis_name="c", subcore_axis_name="s")
    return pl.kernel(
        body,
        compiler_params=pltpu.CompilerParams(needs_layout_passes=False),
        out_type=(
            jax.ShapeDtypeStruct((NC, npad, d), jnp.float32),
            jax.ShapeDtypeStruct((NC, npad), jnp.float32),
        ),
        mesh=mesh,
        scratch_types=[
            pltpu.VMEM((npad,), jnp.float32),
            pltpu.VMEM((npad,), jnp.float32),
            pltpu.VMEM((L,), jnp.float32),
            pltpu.VMEM((B,), jnp.int32),
            pltpu.VMEM((B,), jnp.int32),
            pltpu.VMEM((B,), jnp.int32),
            pltpu.VMEM((B,), jnp.int32),
            pltpu.VMEM((B,), jnp.int32),
            pltpu.VMEM((B,), jnp.int32),
            pltpu.VMEM((B,), jnp.float32),
            pltpu.VMEM((B,), jnp.float32),
            pltpu.VMEM((B, d), jnp.float32),
            pltpu.VMEM((B, d), jnp.float32),
            pltpu.VMEM_SHARED((npad, d), jnp.float32),
            pltpu.VMEM_SHARED((npad,), jnp.float32),
            pltpu.SemaphoreType.DMA,
            pltpu.SemaphoreType.DMA,
            pltpu.SemaphoreType.DMA,
            pltpu.SemaphoreType.DMA,
            pltpu.SemaphoreType.DMA,
            pltpu.SemaphoreType.DMA,
            pltpu.SemaphoreType.DMA,
        ],
    )


def kernel(x, edge_index, W, att_src, att_dst, bias):
    n, d = x.shape
    e = edge_index.shape[1]
    total = e + n
    npad = (n + L + NS * L - 1) // (NS * L) * (NS * L)
    epad = (total + NW * B * 6 - 1) // (NW * B * 6) * (NW * B * 6)

    loop = jnp.arange(n, dtype=jnp.int32)
    src = jnp.concatenate([edge_index[0], loop,
                           jnp.zeros((epad - total,), jnp.int32)])
    dst = jnp.concatenate([edge_index[1], loop,
                           jnp.full((epad - total,), n, jnp.int32)])

    h, a_src, a_dst, c_vec = _tc_pre(x, W, att_src, att_dst, npad)

    zrow = jnp.zeros((npad // NS, d), jnp.float32)
    zvec = jnp.zeros((npad,), jnp.float32)
    acc, den = _sc_edge_kernel(npad, epad, d)(
        src, dst, h, a_src, a_dst, c_vec, zrow, zvec)

    return _tc_post(acc, den, bias, n)


# speedup vs baseline: 1.0573x; 1.0573x over previous
import functools

import jax
import jax.numpy as jnp
from jax import lax
from jax.experimental import pallas as pl
from jax.experimental.pallas import tpu as pltpu
from jax.experimental.pallas import tpu_sc as plsc

NEG_SLOPE = 0.2
NC, NS, L = 2, 16, 16
NW = NC * NS
B = 96
DG = 8
ZR = 16


def _tc_pre(x, W, att_src, att_dst, npad):
    n, d = x.shape

    def body(x_ref, w_ref, s_ref, t_ref, h_ref, as_ref, ad_ref, c_ref):
        h = jnp.dot(x_ref[...], w_ref[...], preferred_element_type=jnp.float32)
        h_ref[...] = h
        a_s = jnp.sum(h * s_ref[...][None, :], axis=1)
        a_d = jnp.sum(h * t_ref[...][None, :], axis=1)
        pad = jnp.zeros((npad - n,), jnp.float32)
        as_ref[...] = jnp.concatenate([a_s, pad])
        ad_ref[...] = jnp.concatenate([a_d, pad])
        c = jnp.maximum(jnp.max(a_s) + jnp.max(a_d), 0.0)
        c_ref[...] = jnp.full((L,), c, jnp.float32)

    return pl.pallas_call(
        body,
        out_shape=(
            jax.ShapeDtypeStruct((n, d), jnp.float32),
            jax.ShapeDtypeStruct((npad,), jnp.float32),
            jax.ShapeDtypeStruct((npad,), jnp.float32),
            jax.ShapeDtypeStruct((L,), jnp.float32),
        ),
    )(x, W, att_src, att_dst)


def _tc_post(acc, den, bias, n):
    def body(a_ref, d_ref, b_ref, o_ref):
        a = a_ref[0] + a_ref[1]
        den_sum = d_ref[0] + d_ref[1] + 1e-16
        o_ref[...] = a[:n] / den_sum[:n, None] + b_ref[...][None, :]

    return pl.pallas_call(
        body,
        out_shape=jax.ShapeDtypeStruct((n, acc.shape[2]), jnp.float32),
    )(acc, den, bias)


def _sc_edge_kernel(npad, epad, d):
    epb = epad // NW
    nch = epb // B
    rpt = npad // NS
    assert nch % 6 == 0 and rpt % 8 == 0 and rpt % ZR == 0

    def body(src_hbm, dst_hbm, h_hbm, asrc_hbm, adst_hbm, c_hbm,
             zvec_hbm, acc_hbm, den_hbm,
             asrc_v, adst_v, c_v, sidx0, sidx1, sidx2, didx0, didx1, didx2,
             p0, p1, rows0, rows1, zbuf, acc_sh, den_sh,
             gs0, gs1, ss0, ss1, is0, is1, is2):
        cid = lax.axis_index("c")
        sid = lax.axis_index("s")
        wid = cid * NS + sid
        sidx = (sidx0, sidx1, sidx2)
        didx = (didx0, didx1, didx2)
        pb = (p0, p1)
        rows = (rows0, rows1)
        gsem = (gs0, gs1)
        ssem = (ss0, ss1)
        isem = (is0, is1, is2)
        def off(g):
            return (g * NW + wid) * B

        def idx_load(g, s3, sync):
            fn = pltpu.sync_copy if sync else (
                lambda s_, d_: pltpu.async_copy(s_, d_, isem[s3]))
            fn(src_hbm.at[pl.ds(off(g), B)], sidx[s3])
            fn(dst_hbm.at[pl.ds(off(g), B)], didx[s3])

        def idx_wait(g, s3):
            pltpu.make_async_copy(
                src_hbm.at[pl.ds(off(g), B)], sidx[s3],
                isem[s3]).wait()
            pltpu.make_async_copy(
                dst_hbm.at[pl.ds(off(g), B)], didx[s3],
                isem[s3]).wait()

        def gather_start(s3, s2):
            pltpu.async_copy(h_hbm.at[sidx[s3]], rows[s2], gsem[s2])

        def gather_wait(s3, s2):
            pltpu.make_async_copy(
                h_hbm.at[sidx[s3]], rows[s2], gsem[s2]).wait()

        def scatter_start(s3, s2):
            pltpu.async_copy(rows[s2], acc_sh.at[didx[s3]],
                             ssem[s2], add=True)
            pltpu.async_copy(pb[s2], den_sh.at[didx[s3]],
                             ssem[s2], add=True)

        def scatter_wait(s3, s2):
            pltpu.make_async_copy(rows[s2], acc_sh.at[didx[s3]],
                                  ssem[s2]).wait()
            pltpu.make_async_copy(pb[s2], den_sh.at[didx[s3]],
                                  ssem[s2]).wait()

        @pl.loop(0, ZR)
        def _zf(r):
            for c in range(d // L):
                zbuf[r, pl.ds(c * L, L)] = jnp.zeros((L,), jnp.float32)

        @pl.loop(0, rpt // ZR)
        def _zc(k):
            pltpu.sync_copy(zbuf, acc_sh.at[pl.ds(sid * rpt + k * ZR, ZR)])

        pltpu.sync_copy(zvec_hbm.at[pl.ds(sid * rpt, rpt)],
                        den_sh.at[pl.ds(sid * rpt, rpt)])
        pltpu.sync_copy(asrc_hbm, asrc_v)
        pltpu.sync_copy(adst_hbm, adst_v)
        pltpu.sync_copy(c_hbm, c_v)
        idx_load(0, 0, True)
        idx_load(1, 1, True)
        gather_start(0, 0)
        gather_start(1, 1)
        idx_load(2, 2, False)
        plsc.subcore_barrier()

        @pl.loop(0, nch // 6)
        def _t(t):
            for i in range(6):
                g = t * 6 + i
                s2, s3 = i % 2, i % 3

                @pl.when(g >= 1)
                def _drain():
                    scatter_wait((i - 1) % 3, (i - 1) % 2)

                @pl.when((g >= 1) & (g + 2 < nch))
                def _pf_idx():
                    idx_load(g + 2, (i + 2) % 3, False)

                @pl.when((g + 1 >= 2) & (g + 1 < nch))
                def _pf_rows():
                    idx_wait(g + 1, (i + 1) % 3)
                    gather_start((i + 1) % 3, (i + 1) % 2)

                gather_wait(s3, s2)

                @plsc.parallel_loop(0, B, step=L, unroll=2)
                def _p(j):
                    j = pl.multiple_of(j, L)
                    sv = sidx[s3][pl.ds(j, L)]
                    dv = didx[s3][pl.ds(j, L)]
                    e = (plsc.load_gather(asrc_v, [sv])
                         + plsc.load_gather(adst_v, [dv]))
                    e = jnp.where(e > 0, e, NEG_SLOPE * e)
                    pb[s2][pl.ds(j, L)] = jnp.exp(e - c_v[...])

                @plsc.parallel_loop(0, B, unroll=4)
                def _scale(j):
                    pj = plsc.load_gather(
                        pb[s2], [jnp.full((L,), 0, jnp.int32) + j])
                    for k in range(DG):
                        rows[s2][j, pl.ds(k * L, L)] = (
                            rows[s2][j, pl.ds(k * L, L)] * pj)

                scatter_start(s3, s2)

        scatter_wait((nch - 1) % 3, (nch - 1) % 2)
        plsc.subcore_barrier()
        pltpu.sync_copy(acc_sh.at[pl.ds(sid * rpt, rpt)],
                        acc_hbm.at[cid, pl.ds(sid * rpt, rpt)])
        pltpu.sync_copy(den_sh.at[pl.ds(sid * rpt, rpt)],
                        den_hbm.at[cid, pl.ds(sid * rpt, rpt)])

    mesh = plsc.VectorSubcoreMesh(core_axis_name="c", subcore_axis_name="s")
    return pl.kernel(
        body,
        compiler_params=pltpu.CompilerParams(needs_layout_passes=False),
        out_type=(
            jax.ShapeDtypeStruct((NC, npad, d), jnp.float32),
            jax.ShapeDtypeStruct((NC, npad), jnp.float32),
        ),
        mesh=mesh,
        scratch_types=[
            pltpu.VMEM((npad,), jnp.float32),
            pltpu.VMEM((npad,), jnp.float32),
            pltpu.VMEM((L,), jnp.float32),
            pltpu.VMEM((B,), jnp.int32),
            pltpu.VMEM((B,), jnp.int32),
            pltpu.VMEM((B,), jnp.int32),
            pltpu.VMEM((B,), jnp.int32),
            pltpu.VMEM((B,), jnp.int32),
            pltpu.VMEM((B,), jnp.int32),
            pltpu.VMEM((B,), jnp.float32),
            pltpu.VMEM((B,), jnp.float32),
            pltpu.VMEM((B, d), jnp.float32),
            pltpu.VMEM((B, d), jnp.float32),
            pltpu.VMEM((ZR, d), jnp.float32),
            pltpu.VMEM_SHARED((npad, d), jnp.float32),
            pltpu.VMEM_SHARED((npad,), jnp.float32),
            pltpu.SemaphoreType.DMA,
            pltpu.SemaphoreType.DMA,
            pltpu.SemaphoreType.DMA,
            pltpu.SemaphoreType.DMA,
            pltpu.SemaphoreType.DMA,
            pltpu.SemaphoreType.DMA,
            pltpu.SemaphoreType.DMA,
        ],
    )


def kernel(x, edge_index, W, att_src, att_dst, bias):
    n, d = x.shape
    e = edge_index.shape[1]
    total = e + n
    npad = (n + L + NS * L - 1) // (NS * L) * (NS * L)
    epad = (total + NW * B * 6 - 1) // (NW * B * 6) * (NW * B * 6)

    loop = jnp.arange(n, dtype=jnp.int32)
    src = jnp.concatenate([edge_index[0], loop,
                           jnp.zeros((epad - total,), jnp.int32)])
    dst = jnp.concatenate([edge_index[1], loop,
                           jnp.full((epad - total,), n, jnp.int32)])

    h, a_src, a_dst, c_vec = _tc_pre(x, W, att_src, att_dst, npad)

    zvec = jnp.zeros((npad,), jnp.float32)
    acc, den = _sc_edge_kernel(npad, epad, d)(
        src, dst, h, a_src, a_dst, c_vec, zvec)

    return _tc_post(acc, den, bias, n)


# speedup vs baseline: 1.0595x; 1.0021x over previous
import functools

import jax
import jax.numpy as jnp
from jax import lax
from jax.experimental import pallas as pl
from jax.experimental.pallas import tpu as pltpu
from jax.experimental.pallas import tpu_sc as plsc

NEG_SLOPE = 0.2
NC, NS, L = 2, 16, 16
NW = NC * NS
B = 96
DG = 8
ZR = 16


def _tc_pre(x, W, att_src, att_dst, npad):
    n, d = x.shape

    def body(x_ref, w_ref, s_ref, t_ref, h_ref, as_ref, ad_ref, c_ref):
        h = jnp.dot(x_ref[...], w_ref[...], preferred_element_type=jnp.float32)
        h_ref[...] = h
        a_s = jnp.sum(h * s_ref[...][None, :], axis=1)
        a_d = jnp.sum(h * t_ref[...][None, :], axis=1)
        pad = jnp.zeros((npad - n,), jnp.float32)
        as_ref[...] = jnp.concatenate([a_s, pad])
        ad_ref[...] = jnp.concatenate([a_d, pad])
        c = jnp.maximum(jnp.max(a_s) + jnp.max(a_d), 0.0)
        c_ref[...] = jnp.full((L,), c, jnp.float32)

    return pl.pallas_call(
        body,
        out_shape=(
            jax.ShapeDtypeStruct((n, d), jnp.float32),
            jax.ShapeDtypeStruct((npad,), jnp.float32),
            jax.ShapeDtypeStruct((npad,), jnp.float32),
            jax.ShapeDtypeStruct((L,), jnp.float32),
        ),
    )(x, W, att_src, att_dst)


def _tc_post(acc, den, bias, n):
    def body(a_ref, d_ref, b_ref, o_ref):
        a = a_ref[0] + a_ref[1]
        den_sum = d_ref[0] + d_ref[1] + 1e-16
        o_ref[...] = a[:n] / den_sum[:n, None] + b_ref[...][None, :]

    return pl.pallas_call(
        body,
        out_shape=jax.ShapeDtypeStruct((n, acc.shape[2]), jnp.float32),
    )(acc, den, bias)


def _sc_edge_kernel(npad, epad, d):
    epb = epad // NW
    nch = epb // B
    rpt = npad // NS
    assert nch % 6 == 0 and rpt % 8 == 0 and rpt % ZR == 0

    def body(src_hbm, dst_hbm, h_hbm, asrc_hbm, adst_hbm, c_hbm, zrow_hbm,
             zvec_hbm, acc_hbm, den_hbm,
             asrc_v, adst_v, c_v, sidx0, sidx1, sidx2, didx0, didx1, didx2,
             p0, p1, rows0, rows1, zbuf, acc_sh, den_sh,
             gs0, gs1, ss0, ss1, is0, is1, is2):
        cid = lax.axis_index("c")
        sid = lax.axis_index("s")
        wid = cid * NS + sid
        sidx = (sidx0, sidx1, sidx2)
        didx = (didx0, didx1, didx2)
        pb = (p0, p1)
        rows = (rows0, rows1)
        gsem = (gs0, gs1)
        ssem = (ss0, ss1)
        isem = (is0, is1, is2)
        def off(g):
            return (g * NW + wid) * B

        def idx_load(g, s3, sync):
            fn = pltpu.sync_copy if sync else (
                lambda s_, d_: pltpu.async_copy(s_, d_, isem[s3]))
            fn(src_hbm.at[pl.ds(off(g), B)], sidx[s3])
            fn(dst_hbm.at[pl.ds(off(g), B)], didx[s3])

        def idx_wait(g, s3):
            pltpu.make_async_copy(
                src_hbm.at[pl.ds(off(g), B)], sidx[s3],
                isem[s3]).wait()
            pltpu.make_async_copy(
                dst_hbm.at[pl.ds(off(g), B)], didx[s3],
                isem[s3]).wait()

        def gather_start(s3, s2):
            pltpu.async_copy(h_hbm.at[sidx[s3]], rows[s2], gsem[s2])

        def gather_wait(s3, s2):
            pltpu.make_async_copy(
                h_hbm.at[sidx[s3]], rows[s2], gsem[s2]).wait()

        def scatter_start(s3, s2):
            pltpu.async_copy(rows[s2], acc_sh.at[didx[s3]],
                             ssem[s2], add=True)
            pltpu.async_copy(pb[s2], den_sh.at[didx[s3]],
                             ssem[s2], add=True)

        def scatter_wait(s3, s2):
            pltpu.make_async_copy(rows[s2], acc_sh.at[didx[s3]],
                                  ssem[s2]).wait()
            pltpu.make_async_copy(pb[s2], den_sh.at[didx[s3]],
                                  ssem[s2]).wait()

        pltpu.sync_copy(zrow_hbm, acc_sh.at[pl.ds(sid * rpt, rpt)])
        pltpu.sync_copy(zvec_hbm.at[pl.ds(sid * rpt, rpt)],
                        den_sh.at[pl.ds(sid * rpt, rpt)])
        pltpu.sync_copy(asrc_hbm, asrc_v)
        pltpu.sync_copy(adst_hbm, adst_v)
        pltpu.sync_copy(c_hbm, c_v)
        idx_load(0, 0, True)
        idx_load(1, 1, True)
        gather_start(0, 0)
        gather_start(1, 1)
        idx_load(2, 2, False)
        plsc.subcore_barrier()

        @pl.loop(0, nch // 6)
        def _t(t):
            for i in range(6):
                g = t * 6 + i
                s2, s3 = i % 2, i % 3

                @pl.when(g >= 1)
                def _drain():
                    scatter_wait((i - 1) % 3, (i - 1) % 2)

                @pl.when((g >= 1) & (g + 2 < nch))
                def _pf_idx():
                    idx_load(g + 2, (i + 2) % 3, False)

                @pl.when((g + 1 >= 2) & (g + 1 < nch))
                def _pf_rows():
                    idx_wait(g + 1, (i + 1) % 3)
                    gather_start((i + 1) % 3, (i + 1) % 2)

                gather_wait(s3, s2)

                @plsc.parallel_loop(0, B, step=L, unroll=2)
                def _p(j):
                    j = pl.multiple_of(j, L)
                    sv = sidx[s3][pl.ds(j, L)]
                    dv = didx[s3][pl.ds(j, L)]
                    e = (plsc.load_gather(asrc_v, [sv])
                         + plsc.load_gather(adst_v, [dv]))
                    e = jnp.where(e > 0, e, NEG_SLOPE * e)
                    pb[s2][pl.ds(j, L)] = jnp.exp(e - c_v[...])

                @plsc.parallel_loop(0, B, unroll=4)
                def _scale(j):
                    pj = plsc.load_gather(
                        pb[s2], [jnp.full((L,), 0, jnp.int32) + j])
                    for k in range(DG):
                        rows[s2][j, pl.ds(k * L, L)] = (
                            rows[s2][j, pl.ds(k * L, L)] * pj)

                scatter_start(s3, s2)

        scatter_wait((nch - 1) % 3, (nch - 1) % 2)
        plsc.subcore_barrier()
        pltpu.sync_copy(acc_sh.at[pl.ds(sid * rpt, rpt)],
                        acc_hbm.at[cid, pl.ds(sid * rpt, rpt)])
        pltpu.sync_copy(den_sh.at[pl.ds(sid * rpt, rpt)],
                        den_hbm.at[cid, pl.ds(sid * rpt, rpt)])

    mesh = plsc.VectorSubcoreMesh(core_axis_name="c", subcore_axis_name="s")
    return pl.kernel(
        body,
        compiler_params=pltpu.CompilerParams(needs_layout_passes=False),
        out_type=(
            jax.ShapeDtypeStruct((NC, npad, d), jnp.float32),
            jax.ShapeDtypeStruct((NC, npad), jnp.float32),
        ),
        mesh=mesh,
        scratch_types=[
            pltpu.VMEM((npad,), jnp.float32),
            pltpu.VMEM((npad,), jnp.float32),
            pltpu.VMEM((L,), jnp.float32),
            pltpu.VMEM((B,), jnp.int32),
            pltpu.VMEM((B,), jnp.int32),
            pltpu.VMEM((B,), jnp.int32),
            pltpu.VMEM((B,), jnp.int32),
            pltpu.VMEM((B,), jnp.int32),
            pltpu.VMEM((B,), jnp.int32),
            pltpu.VMEM((B,), jnp.float32),
            pltpu.VMEM((B,), jnp.float32),
            pltpu.VMEM((B, d), jnp.float32),
            pltpu.VMEM((B, d), jnp.float32),
            pltpu.VMEM((ZR, d), jnp.float32),
            pltpu.VMEM_SHARED((npad, d), jnp.float32),
            pltpu.VMEM_SHARED((npad,), jnp.float32),
            pltpu.SemaphoreType.DMA,
            pltpu.SemaphoreType.DMA,
            pltpu.SemaphoreType.DMA,
            pltpu.SemaphoreType.DMA,
            pltpu.SemaphoreType.DMA,
            pltpu.SemaphoreType.DMA,
            pltpu.SemaphoreType.DMA,
        ],
    )


def kernel(x, edge_index, W, att_src, att_dst, bias):
    n, d = x.shape
    e = edge_index.shape[1]
    total = e + n
    npad = (n + L + NS * L - 1) // (NS * L) * (NS * L)
    epad = (total + NW * B * 6 - 1) // (NW * B * 6) * (NW * B * 6)

    loop = jnp.arange(n, dtype=jnp.int32)
    src = jnp.concatenate([edge_index[0], loop,
                           jnp.zeros((epad - total,), jnp.int32)])
    dst = jnp.concatenate([edge_index[1], loop,
                           jnp.full((epad - total,), n, jnp.int32)])

    h, a_src, a_dst, c_vec = _tc_pre(x, W, att_src, att_dst, npad)

    zrow = jnp.zeros((npad // NS, d), jnp.float32)
    zvec = jnp.zeros((npad,), jnp.float32)
    acc, den = _sc_edge_kernel(npad, epad, d)(
        src, dst, h, a_src, a_dst, c_vec, zrow, zvec)

    return _tc_post(acc, den, bias, n)


# speedup vs baseline: 1.0726x; 1.0124x over previous
import functools

import jax
import jax.numpy as jnp
from jax import lax
from jax.experimental import pallas as pl
from jax.experimental.pallas import tpu as pltpu
from jax.experimental.pallas import tpu_sc as plsc

NEG_SLOPE = 0.2
NC, NS, L = 2, 16, 16
NW = NC * NS
B = 96
DG = 8


def _tc_pre(x, W, att_src, att_dst, npad):
    n, d = x.shape

    def body(x_ref, w_ref, s_ref, t_ref, h_ref, as_ref, ad_ref, c_ref):
        h = jnp.dot(x_ref[...], w_ref[...], preferred_element_type=jnp.float32)
        h_ref[...] = h
        a_s = jnp.sum(h * s_ref[...][None, :], axis=1)
        a_d = jnp.sum(h * t_ref[...][None, :], axis=1)
        pad = jnp.zeros((npad - n,), jnp.float32)
        as_ref[...] = jnp.concatenate([a_s, pad])
        ad_ref[...] = jnp.concatenate([a_d, pad])
        c = jnp.maximum(jnp.max(a_s) + jnp.max(a_d), 0.0)
        c_ref[...] = jnp.full((L,), c, jnp.float32)

    return pl.pallas_call(
        body,
        out_shape=(
            jax.ShapeDtypeStruct((n, d), jnp.float32),
            jax.ShapeDtypeStruct((npad,), jnp.float32),
            jax.ShapeDtypeStruct((npad,), jnp.float32),
            jax.ShapeDtypeStruct((L,), jnp.float32),
        ),
    )(x, W, att_src, att_dst)


def _tc_post(acc, den, bias, n):
    def body(a_ref, d_ref, b_ref, o_ref):
        a = a_ref[0] + a_ref[1]
        den_sum = d_ref[0] + d_ref[1] + 1e-16
        o_ref[...] = a[:n] / den_sum[:n, None] + b_ref[...][None, :]

    return pl.pallas_call(
        body,
        out_shape=jax.ShapeDtypeStruct((n, acc.shape[2]), jnp.float32),
    )(acc, den, bias)


def _sc_edge_kernel(npad, epad, d):
    epb = epad // NW
    nch = epb // B
    rpt = npad // NS
    assert nch % 6 == 0 and rpt % 8 == 0

    def body(pk_hbm, h_hbm, asrc_hbm, adst_hbm, c_hbm, zrow_hbm,
             zvec_hbm, acc_hbm, den_hbm,
             asrc_v, adst_v, c_v, pk0, pk1, pk2,
             sidx0, sidx1, sidx2, didx0, didx1, didx2,
             p0, p1, rows0, rows1, acc_sh, den_sh,
             gs0, gs1, ss0, ss1, is0, is1, is2):
        cid = lax.axis_index("c")
        sid = lax.axis_index("s")
        wid = cid * NS + sid
        pk = (pk0, pk1, pk2)
        sidx = (sidx0, sidx1, sidx2)
        didx = (didx0, didx1, didx2)
        pb = (p0, p1)
        rows = (rows0, rows1)
        gsem = (gs0, gs1)
        ssem = (ss0, ss1)
        isem = (is0, is1, is2)
        def off(g):
            return (g * NW + wid) * B

        def idx_load(g, s3, sync):
            fn = pltpu.sync_copy if sync else (
                lambda s_, d_: pltpu.async_copy(s_, d_, isem[s3]))
            fn(pk_hbm.at[pl.ds(off(g), B)], pk[s3])

        def idx_wait(g, s3):
            pltpu.make_async_copy(
                pk_hbm.at[pl.ds(off(g), B)], pk[s3], isem[s3]).wait()

        def idx_unpack(s3):
            @plsc.parallel_loop(0, B, step=L, unroll=2)
            def _u(j):
                j = pl.multiple_of(j, L)
                v = pk[s3][pl.ds(j, L)]
                sidx[s3][pl.ds(j, L)] = v & jnp.int32(0xFFFF)
                didx[s3][pl.ds(j, L)] = jax.lax.shift_right_logical(
                    v, jnp.int32(16))

        def gather_start(s3, s2):
            pltpu.async_copy(h_hbm.at[sidx[s3]], rows[s2], gsem[s2])

        def gather_wait(s3, s2):
            pltpu.make_async_copy(
                h_hbm.at[sidx[s3]], rows[s2], gsem[s2]).wait()

        def scatter_start(s3, s2):
            pltpu.async_copy(rows[s2], acc_sh.at[didx[s3]],
                             ssem[s2], add=True)
            pltpu.async_copy(pb[s2], den_sh.at[didx[s3]],
                             ssem[s2], add=True)

        def scatter_wait(s3, s2):
            pltpu.make_async_copy(rows[s2], acc_sh.at[didx[s3]],
                                  ssem[s2]).wait()
            pltpu.make_async_copy(pb[s2], den_sh.at[didx[s3]],
                                  ssem[s2]).wait()

        pltpu.sync_copy(zrow_hbm, acc_sh.at[pl.ds(sid * rpt, rpt)])
        pltpu.sync_copy(zvec_hbm.at[pl.ds(sid * rpt, rpt)],
                        den_sh.at[pl.ds(sid * rpt, rpt)])
        pltpu.sync_copy(asrc_hbm, asrc_v)
        pltpu.sync_copy(adst_hbm, adst_v)
        pltpu.sync_copy(c_hbm, c_v)
        idx_load(0, 0, True)
        idx_load(1, 1, True)
        idx_unpack(0)
        idx_unpack(1)
        gather_start(0, 0)
        gather_start(1, 1)
        idx_load(2, 2, False)
        plsc.subcore_barrier()

        @pl.loop(0, nch // 6)
        def _t(t):
            for i in range(6):
                g = t * 6 + i
                s2, s3 = i % 2, i % 3

                @pl.when(g >= 1)
                def _drain():
                    scatter_wait((i - 1) % 3, (i - 1) % 2)

                @pl.when((g >= 1) & (g + 2 < nch))
                def _pf_idx():
                    idx_load(g + 2, (i + 2) % 3, False)

                @pl.when((g + 1 >= 2) & (g + 1 < nch))
                def _pf_rows():
                    idx_wait(g + 1, (i + 1) % 3)
                    idx_unpack((i + 1) % 3)
                    gather_start((i + 1) % 3, (i + 1) % 2)

                gather_wait(s3, s2)

                @plsc.parallel_loop(0, B, step=L, unroll=2)
                def _p(j):
                    j = pl.multiple_of(j, L)
                    sv = sidx[s3][pl.ds(j, L)]
                    dv = didx[s3][pl.ds(j, L)]
                    e = (plsc.load_gather(asrc_v, [sv])
                         + plsc.load_gather(adst_v, [dv]))
                    e = jnp.where(e > 0, e, NEG_SLOPE * e)
                    pb[s2][pl.ds(j, L)] = jnp.exp(e - c_v[...])

                @plsc.parallel_loop(0, B, unroll=4)
                def _scale(j):
                    pj = plsc.load_gather(
                        pb[s2], [jnp.full((L,), 0, jnp.int32) + j])
                    for k in range(DG):
                        rows[s2][j, pl.ds(k * L, L)] = (
                            rows[s2][j, pl.ds(k * L, L)] * pj)

                scatter_start(s3, s2)

        scatter_wait((nch - 1) % 3, (nch - 1) % 2)
        plsc.subcore_barrier()
        pltpu.sync_copy(acc_sh.at[pl.ds(sid * rpt, rpt)],
                        acc_hbm.at[cid, pl.ds(sid * rpt, rpt)])
        pltpu.sync_copy(den_sh.at[pl.ds(sid * rpt, rpt)],
                        den_hbm.at[cid, pl.ds(sid * rpt, rpt)])

    mesh = plsc.VectorSubcoreMesh(core_axis_name="c", subcore_axis_name="s")
    return pl.kernel(
        body,
        compiler_params=pltpu.CompilerParams(needs_layout_passes=False),
        out_type=(
            jax.ShapeDtypeStruct((NC, npad, d), jnp.float32),
            jax.ShapeDtypeStruct((NC, npad), jnp.float32),
        ),
        mesh=mesh,
        scratch_types=[
            pltpu.VMEM((npad,), jnp.float32),
            pltpu.VMEM((npad,), jnp.float32),
            pltpu.VMEM((L,), jnp.float32),
            pltpu.VMEM((B,), jnp.int32),
            pltpu.VMEM((B,), jnp.int32),
            pltpu.VMEM((B,), jnp.int32),
            pltpu.VMEM((B,), jnp.int32),
            pltpu.VMEM((B,), jnp.int32),
            pltpu.VMEM((B,), jnp.int32),
            pltpu.VMEM((B,), jnp.int32),
            pltpu.VMEM((B,), jnp.int32),
            pltpu.VMEM((B,), jnp.int32),
            pltpu.VMEM((B,), jnp.float32),
            pltpu.VMEM((B,), jnp.float32),
            pltpu.VMEM((B, d), jnp.float32),
            pltpu.VMEM((B, d), jnp.float32),
            pltpu.VMEM_SHARED((npad, d), jnp.float32),
            pltpu.VMEM_SHARED((npad,), jnp.float32),
            pltpu.SemaphoreType.DMA,
            pltpu.SemaphoreType.DMA,
            pltpu.SemaphoreType.DMA,
            pltpu.SemaphoreType.DMA,
            pltpu.SemaphoreType.DMA,
            pltpu.SemaphoreType.DMA,
            pltpu.SemaphoreType.DMA,
        ],
    )


def kernel(x, edge_index, W, att_src, att_dst, bias):
    n, d = x.shape
    e = edge_index.shape[1]
    total = e + n
    npad = (n + L + NS * L - 1) // (NS * L) * (NS * L)
    epad = (total + NW * B * 6 - 1) // (NW * B * 6) * (NW * B * 6)

    loop = jnp.arange(n, dtype=jnp.int32)
    src = jnp.concatenate([edge_index[0], loop,
                           jnp.zeros((epad - total,), jnp.int32)])
    dst = jnp.concatenate([edge_index[1], loop,
                           jnp.full((epad - total,), n, jnp.int32)])
    packed = src | (dst << 16)

    h, a_src, a_dst, c_vec = _tc_pre(x, W, att_src, att_dst, npad)

    zrow = jnp.zeros((npad // NS, d), jnp.float32)
    zvec = jnp.zeros((npad,), jnp.float32)
    acc, den = _sc_edge_kernel(npad, epad, d)(
        packed, h, a_src, a_dst, c_vec, zrow, zvec)

    return _tc_post(acc, den, bias, n)


# speedup vs baseline: 1.0808x; 1.0076x over previous
import functools

import jax
import jax.numpy as jnp
from jax import lax
from jax.experimental import pallas as pl
from jax.experimental.pallas import tpu as pltpu
from jax.experimental.pallas import tpu_sc as plsc

NEG_SLOPE = 0.2
NC, NS, L = 2, 16, 16
NW = NC * NS
B = 96
DG = 8


def _tc_pre(x, W, att_src, att_dst, npad):
    n, d = x.shape

    def body(x_ref, w_ref, s_ref, t_ref, h_ref, as_ref, ad_ref, c_ref):
        h = jnp.dot(x_ref[...], w_ref[...], preferred_element_type=jnp.float32)
        h_ref[...] = h
        a_s = jnp.sum(h * s_ref[...][None, :], axis=1)
        a_d = jnp.sum(h * t_ref[...][None, :], axis=1)
        pad = jnp.zeros((npad - n,), jnp.float32)
        as_ref[...] = jnp.concatenate([a_s, pad])
        ad_ref[...] = jnp.concatenate([a_d, pad])
        c = jnp.maximum(jnp.max(a_s) + jnp.max(a_d), 0.0)
        c_ref[...] = jnp.full((L,), c, jnp.float32)

    return pl.pallas_call(
        body,
        out_shape=(
            jax.ShapeDtypeStruct((n, d), jnp.float32),
            jax.ShapeDtypeStruct((npad,), jnp.float32),
            jax.ShapeDtypeStruct((npad,), jnp.float32),
            jax.ShapeDtypeStruct((L,), jnp.float32),
        ),
    )(x, W, att_src, att_dst)


def _tc_post(acc, den, bias, n):
    def body(a_ref, d_ref, b_ref, o_ref):
        a = a_ref[0] + a_ref[1]
        den_sum = d_ref[0] + d_ref[1] + 1e-16
        o_ref[...] = a[:n] / den_sum[:n, None] + b_ref[...][None, :]

    return pl.pallas_call(
        body,
        out_shape=jax.ShapeDtypeStruct((n, acc.shape[2]), jnp.float32),
    )(acc, den, bias)


def _sc_edge_kernel(npad, epad, d):
    epb = epad // NW
    nch = epb // B
    rpt = npad // NS
    assert nch % 6 == 0 and rpt % 8 == 0

    def body(pk_hbm, h_hbm, asrc_hbm, adst_hbm, c_hbm, zrow_hbm,
             zvec_hbm, acc_hbm, den_hbm,
             asrc_v, adst_v, c_v, pk0, pk1, pk2,
             sidx0, sidx1, sidx2, didx0, didx1, didx2,
             p0, p1, rows0, rows1, acc_sh, den_sh,
             gs0, gs1, ss0, ss1, is0, is1, is2):
        cid = lax.axis_index("c")
        sid = lax.axis_index("s")
        wid = cid * NS + sid
        pk = (pk0, pk1, pk2)
        sidx = (sidx0, sidx1, sidx2)
        didx = (didx0, didx1, didx2)
        pb = (p0, p1)
        rows = (rows0, rows1)
        gsem = (gs0, gs1)
        ssem = (ss0, ss1)
        isem = (is0, is1, is2)
        def off(g):
            return (g * NW + wid) * B

        def idx_load(g, s3, sync):
            fn = pltpu.sync_copy if sync else (
                lambda s_, d_: pltpu.async_copy(s_, d_, isem[s3]))
            fn(pk_hbm.at[pl.ds(off(g), B)], pk[s3])

        def idx_wait(g, s3):
            pltpu.make_async_copy(
                pk_hbm.at[pl.ds(off(g), B)], pk[s3], isem[s3]).wait()

        def idx_unpack(s3):
            @plsc.parallel_loop(0, B, step=L, unroll=2)
            def _u(j):
                j = pl.multiple_of(j, L)
                v = pk[s3][pl.ds(j, L)]
                sidx[s3][pl.ds(j, L)] = v & jnp.int32(0xFFFF)
                didx[s3][pl.ds(j, L)] = jax.lax.shift_right_logical(
                    v, jnp.int32(16))

        def gather_start(s3, s2):
            pltpu.async_copy(h_hbm.at[sidx[s3]], rows[s2], gsem[s2])

        def gather_wait(s3, s2):
            pltpu.make_async_copy(
                h_hbm.at[sidx[s3]], rows[s2], gsem[s2]).wait()

        def scatter_start(s3, s2):
            pltpu.async_copy(rows[s2], acc_sh.at[didx[s3]],
                             ssem[s2], add=True)
            pltpu.async_copy(pb[s2], den_sh.at[didx[s3]],
                             ssem[s2], add=True)

        def scatter_wait(s3, s2):
            pltpu.make_async_copy(rows[s2], acc_sh.at[didx[s3]],
                                  ssem[s2]).wait()
            pltpu.make_async_copy(pb[s2], den_sh.at[didx[s3]],
                                  ssem[s2]).wait()

        for c in range(d // L):
            zrow0 = jnp.zeros((L,), jnp.float32)
            rows0[0, pl.ds(c * L, L)] = zrow0
        @pl.loop(0, B)
        def _zr(r):
            for c in range(d // L):
                rows0[r, pl.ds(c * L, L)] = rows0[0, pl.ds(c * L, L)] * 0.0

        for k in range(rpt // B):
            pltpu.sync_copy(rows0, acc_sh.at[pl.ds(sid * rpt + k * B, B)])
        if rpt % B:
            pltpu.sync_copy(
                rows0.at[pl.ds(0, rpt % B)],
                acc_sh.at[pl.ds(sid * rpt + (rpt // B) * B, rpt % B)])
        pltpu.sync_copy(zvec_hbm.at[pl.ds(sid * rpt, rpt)],
                        den_sh.at[pl.ds(sid * rpt, rpt)])
        pltpu.sync_copy(asrc_hbm, asrc_v)
        pltpu.sync_copy(adst_hbm, adst_v)
        pltpu.sync_copy(c_hbm, c_v)
        idx_load(0, 0, True)
        idx_load(1, 1, True)
        idx_unpack(0)
        idx_unpack(1)
        gather_start(0, 0)
        gather_start(1, 1)
        idx_load(2, 2, False)
        plsc.subcore_barrier()

        @pl.loop(0, nch // 6)
        def _t(t):
            for i in range(6):
                g = t * 6 + i
                s2, s3 = i % 2, i % 3

                @pl.when(g >= 1)
                def _drain():
                    scatter_wait((i - 1) % 3, (i - 1) % 2)

                @pl.when((g >= 1) & (g + 2 < nch))
                def _pf_idx():
                    idx_load(g + 2, (i + 2) % 3, False)

                @pl.when((g + 1 >= 2) & (g + 1 < nch))
                def _pf_rows():
                    idx_wait(g + 1, (i + 1) % 3)
                    idx_unpack((i + 1) % 3)
                    gather_start((i + 1) % 3, (i + 1) % 2)

                gather_wait(s3, s2)

                @plsc.parallel_loop(0, B, step=L, unroll=2)
                def _p(j):
                    j = pl.multiple_of(j, L)
                    sv = sidx[s3][pl.ds(j, L)]
                    dv = didx[s3][pl.ds(j, L)]
                    e = (plsc.load_gather(asrc_v, [sv])
                         + plsc.load_gather(adst_v, [dv]))
                    e = jnp.where(e > 0, e, NEG_SLOPE * e)
                    pb[s2][pl.ds(j, L)] = jnp.exp(e - c_v[...])

                @plsc.parallel_loop(0, B, unroll=4)
                def _scale(j):
                    pj = plsc.load_gather(
                        pb[s2], [jnp.full((L,), 0, jnp.int32) + j])
                    for k in range(DG):
                        rows[s2][j, pl.ds(k * L, L)] = (
                            rows[s2][j, pl.ds(k * L, L)] * pj)

                scatter_start(s3, s2)

        scatter_wait((nch - 1) % 3, (nch - 1) % 2)
        plsc.subcore_barrier()
        pltpu.sync_copy(acc_sh.at[pl.ds(sid * rpt, rpt)],
                        acc_hbm.at[cid, pl.ds(sid * rpt, rpt)])
        pltpu.sync_copy(den_sh.at[pl.ds(sid * rpt, rpt)],
                        den_hbm.at[cid, pl.ds(sid * rpt, rpt)])

    mesh = plsc.VectorSubcoreMesh(core_axis_name="c", subcore_axis_name="s")
    return pl.kernel(
        body,
        compiler_params=pltpu.CompilerParams(needs_layout_passes=False),
        out_type=(
            jax.ShapeDtypeStruct((NC, npad, d), jnp.float32),
            jax.ShapeDtypeStruct((NC, npad), jnp.float32),
        ),
        mesh=mesh,
        scratch_types=[
            pltpu.VMEM((npad,), jnp.float32),
            pltpu.VMEM((npad,), jnp.float32),
            pltpu.VMEM((L,), jnp.float32),
            pltpu.VMEM((B,), jnp.int32),
            pltpu.VMEM((B,), jnp.int32),
            pltpu.VMEM((B,), jnp.int32),
            pltpu.VMEM((B,), jnp.int32),
            pltpu.VMEM((B,), jnp.int32),
            pltpu.VMEM((B,), jnp.int32),
            pltpu.VMEM((B,), jnp.int32),
            pltpu.VMEM((B,), jnp.int32),
            pltpu.VMEM((B,), jnp.int32),
            pltpu.VMEM((B,), jnp.float32),
            pltpu.VMEM((B,), jnp.float32),
            pltpu.VMEM((B, d), jnp.float32),
            pltpu.VMEM((B, d), jnp.float32),
            pltpu.VMEM_SHARED((npad, d), jnp.float32),
            pltpu.VMEM_SHARED((npad,), jnp.float32),
            pltpu.SemaphoreType.DMA,
            pltpu.SemaphoreType.DMA,
            pltpu.SemaphoreType.DMA,
            pltpu.SemaphoreType.DMA,
            pltpu.SemaphoreType.DMA,
            pltpu.SemaphoreType.DMA,
            pltpu.SemaphoreType.DMA,
        ],
    )


def kernel(x, edge_index, W, att_src, att_dst, bias):
    n, d = x.shape
    e = edge_index.shape[1]
    total = e + n
    npad = (n + L + NS * L - 1) // (NS * L) * (NS * L)
    epad = (total + NW * B * 6 - 1) // (NW * B * 6) * (NW * B * 6)

    loop = jnp.arange(n, dtype=jnp.int32)
    src = jnp.concatenate([edge_index[0], loop,
                           jnp.zeros((epad - total,), jnp.int32)])
    dst = jnp.concatenate([edge_index[1], loop,
                           jnp.full((epad - total,), n, jnp.int32)])
    packed = src | (dst << 16)

    h, a_src, a_dst, c_vec = _tc_pre(x, W, att_src, att_dst, npad)

    zrow = jnp.zeros((npad // NS, d), jnp.float32)
    zvec = jnp.zeros((npad,), jnp.float32)
    acc, den = _sc_edge_kernel(npad, epad, d)(
        packed, h, a_src, a_dst, c_vec, zrow, zvec)

    return _tc_post(acc, den, bias, n)


# speedup vs baseline: 1.0808x; 1.0001x over previous
import jax
import jax.numpy as jnp
from jax import lax
from jax.experimental import pallas as pl
from jax.experimental.pallas import tpu as pltpu
from jax.experimental.pallas import tpu_sc as plsc

NEG_SLOPE = 0.2
NC, NS, L = 2, 16, 16
NW = NC * NS
B = 96
DG = 8


def _tc_pre(x, W, att_src, att_dst, npad):
    n, d = x.shape

    def body(x_ref, w_ref, s_ref, t_ref, h_ref, as_ref, ad_ref, c_ref):
        h = jnp.dot(x_ref[...], w_ref[...], preferred_element_type=jnp.float32)
        h_ref[...] = h
        a_s = jnp.sum(h * s_ref[...][None, :], axis=1)
        a_d = jnp.sum(h * t_ref[...][None, :], axis=1)
        pad = jnp.zeros((npad - n,), jnp.float32)
        as_ref[...] = jnp.concatenate([a_s, pad])
        ad_ref[...] = jnp.concatenate([a_d, pad])
        c = jnp.maximum(jnp.max(a_s) + jnp.max(a_d), 0.0)
        c_ref[...] = jnp.full((L,), c, jnp.float32)

    return pl.pallas_call(
        body,
        out_shape=(
            jax.ShapeDtypeStruct((n, d), jnp.float32),
            jax.ShapeDtypeStruct((npad,), jnp.float32),
            jax.ShapeDtypeStruct((npad,), jnp.float32),
            jax.ShapeDtypeStruct((L,), jnp.float32),
        ),
    )(x, W, att_src, att_dst)


def _tc_post(acc, den, bias, n):
    def body(a_ref, d_ref, b_ref, o_ref):
        a = a_ref[0] + a_ref[1]
        den_sum = d_ref[0] + d_ref[1] + 1e-16
        o_ref[...] = a[:n] / den_sum[:n, None] + b_ref[...][None, :]

    return pl.pallas_call(
        body,
        out_shape=jax.ShapeDtypeStruct((n, acc.shape[2]), jnp.float32),
    )(acc, den, bias)


def _sc_edge_kernel(npad, epad, d):
    epb = epad // NW
    nch = epb // B
    rpt = npad // NS
    assert nch % 6 == 0 and rpt % 8 == 0

    def body(pk_hbm, h_hbm, asrc_hbm, adst_hbm, c_hbm, zrow_hbm,
             zvec_hbm, acc_hbm, den_hbm,
             asrc_v, adst_v, c_v, pk0, pk1, pk2,
             sidx0, sidx1, sidx2, didx0, didx1, didx2,
             p0, p1, rows0, rows1, acc_sh, den_sh,
             gs0, gs1, ss0, ss1, is0, is1, is2):
        cid = lax.axis_index("c")
        sid = lax.axis_index("s")
        wid = cid * NS + sid
        pk = (pk0, pk1, pk2)
        sidx = (sidx0, sidx1, sidx2)
        didx = (didx0, didx1, didx2)
        pb = (p0, p1)
        rows = (rows0, rows1)
        gsem = (gs0, gs1)
        ssem = (ss0, ss1)
        isem = (is0, is1, is2)
        def off(g):
            return (g * NW + wid) * B

        def idx_load(g, s3, sync):
            fn = pltpu.sync_copy if sync else (
                lambda s_, d_: pltpu.async_copy(s_, d_, isem[s3]))
            fn(pk_hbm.at[pl.ds(off(g), B)], pk[s3])

        def idx_wait(g, s3):
            pltpu.make_async_copy(
                pk_hbm.at[pl.ds(off(g), B)], pk[s3], isem[s3]).wait()

        def idx_unpack(s3):
            @plsc.parallel_loop(0, B, step=L, unroll=2)
            def _u(j):
                j = pl.multiple_of(j, L)
                v = pk[s3][pl.ds(j, L)]
                sidx[s3][pl.ds(j, L)] = v & jnp.int32(0xFFFF)
                didx[s3][pl.ds(j, L)] = jax.lax.shift_right_logical(
                    v, jnp.int32(16))

        def gather_start(s3, s2):
            pltpu.async_copy(h_hbm.at[sidx[s3]], rows[s2], gsem[s2])

        def gather_wait(s3, s2):
            pltpu.make_async_copy(
                h_hbm.at[sidx[s3]], rows[s2], gsem[s2]).wait()

        def scatter_start(s3, s2):
            pltpu.async_copy(rows[s2], acc_sh.at[didx[s3]],
                             ssem[s2], add=True)
            pltpu.async_copy(pb[s2], den_sh.at[didx[s3]],
                             ssem[s2], add=True)

        def scatter_wait(s3, s2):
            pltpu.make_async_copy(rows[s2], acc_sh.at[didx[s3]],
                                  ssem[s2]).wait()
            pltpu.make_async_copy(pb[s2], den_sh.at[didx[s3]],
                                  ssem[s2]).wait()

        pltpu.sync_copy(zrow_hbm, acc_sh.at[pl.ds(sid * rpt, rpt)])
        pltpu.sync_copy(zvec_hbm.at[pl.ds(sid * rpt, rpt)],
                        den_sh.at[pl.ds(sid * rpt, rpt)])
        pltpu.sync_copy(asrc_hbm, asrc_v)
        pltpu.sync_copy(adst_hbm, adst_v)
        pltpu.sync_copy(c_hbm, c_v)
        idx_load(0, 0, True)
        idx_load(1, 1, True)
        idx_unpack(0)
        idx_unpack(1)
        gather_start(0, 0)
        gather_start(1, 1)
        idx_load(2, 2, False)
        plsc.subcore_barrier()

        @pl.loop(0, nch // 6)
        def _t(t):
            for i in range(6):
                g = t * 6 + i
                s2, s3 = i % 2, i % 3

                @pl.when(g >= 1)
                def _drain():
                    scatter_wait((i - 1) % 3, (i - 1) % 2)

                @pl.when((g >= 1) & (g + 2 < nch))
                def _pf_idx():
                    idx_load(g + 2, (i + 2) % 3, False)

                @pl.when((g + 1 >= 2) & (g + 1 < nch))
                def _pf_rows():
                    idx_wait(g + 1, (i + 1) % 3)
                    idx_unpack((i + 1) % 3)
                    gather_start((i + 1) % 3, (i + 1) % 2)

                gather_wait(s3, s2)

                @plsc.parallel_loop(0, B, step=L, unroll=2)
                def _p(j):
                    j = pl.multiple_of(j, L)
                    sv = sidx[s3][pl.ds(j, L)]
                    dv = didx[s3][pl.ds(j, L)]
                    e = (plsc.load_gather(asrc_v, [sv])
                         + plsc.load_gather(adst_v, [dv]))
                    e = jnp.where(e > 0, e, NEG_SLOPE * e)
                    pb[s2][pl.ds(j, L)] = jnp.exp(e - c_v[...])

                @plsc.parallel_loop(0, B, unroll=4)
                def _scale(j):
                    pj = plsc.load_gather(
                        pb[s2], [jnp.full((L,), 0, jnp.int32) + j])
                    for k in range(DG):
                        rows[s2][j, pl.ds(k * L, L)] = (
                            rows[s2][j, pl.ds(k * L, L)] * pj)

                scatter_start(s3, s2)

        scatter_wait((nch - 1) % 3, (nch - 1) % 2)
        plsc.subcore_barrier()
        pltpu.sync_copy(acc_sh.at[pl.ds(sid * rpt, rpt)],
                        acc_hbm.at[cid, pl.ds(sid * rpt, rpt)])
        pltpu.sync_copy(den_sh.at[pl.ds(sid * rpt, rpt)],
                        den_hbm.at[cid, pl.ds(sid * rpt, rpt)])

    mesh = plsc.VectorSubcoreMesh(core_axis_name="c", subcore_axis_name="s")
    return pl.kernel(
        body,
        compiler_params=pltpu.CompilerParams(needs_layout_passes=False),
        out_type=(
            jax.ShapeDtypeStruct((NC, npad, d), jnp.float32),
            jax.ShapeDtypeStruct((NC, npad), jnp.float32),
        ),
        mesh=mesh,
        scratch_types=[
            pltpu.VMEM((npad,), jnp.float32),
            pltpu.VMEM((npad,), jnp.float32),
            pltpu.VMEM((L,), jnp.float32),
            pltpu.VMEM((B,), jnp.int32),
            pltpu.VMEM((B,), jnp.int32),
            pltpu.VMEM((B,), jnp.int32),
            pltpu.VMEM((B,), jnp.int32),
            pltpu.VMEM((B,), jnp.int32),
            pltpu.VMEM((B,), jnp.int32),
            pltpu.VMEM((B,), jnp.int32),
            pltpu.VMEM((B,), jnp.int32),
            pltpu.VMEM((B,), jnp.int32),
            pltpu.VMEM((B,), jnp.float32),
            pltpu.VMEM((B,), jnp.float32),
            pltpu.VMEM((B, d), jnp.float32),
            pltpu.VMEM((B, d), jnp.float32),
            pltpu.VMEM_SHARED((npad, d), jnp.float32),
            pltpu.VMEM_SHARED((npad,), jnp.float32),
            pltpu.SemaphoreType.DMA,
            pltpu.SemaphoreType.DMA,
            pltpu.SemaphoreType.DMA,
            pltpu.SemaphoreType.DMA,
            pltpu.SemaphoreType.DMA,
            pltpu.SemaphoreType.DMA,
            pltpu.SemaphoreType.DMA,
        ],
    )


def kernel(x, edge_index, W, att_src, att_dst, bias):
    n, d = x.shape
    e = edge_index.shape[1]
    total = e + n
    npad = (n + L + NS * L - 1) // (NS * L) * (NS * L)
    epad = (total + NW * B * 6 - 1) // (NW * B * 6) * (NW * B * 6)

    loop = jnp.arange(n, dtype=jnp.int32)
    src = jnp.concatenate([edge_index[0], loop,
                           jnp.zeros((epad - total,), jnp.int32)])
    dst = jnp.concatenate([edge_index[1], loop,
                           jnp.full((epad - total,), n, jnp.int32)])
    packed = src | (dst << 16)

    h, a_src, a_dst, c_vec = _tc_pre(x, W, att_src, att_dst, npad)

    zrow = jnp.zeros((npad // NS, d), jnp.float32)
    zvec = jnp.zeros((npad,), jnp.float32)
    acc, den = _sc_edge_kernel(npad, epad, d)(
        packed, h, a_src, a_dst, c_vec, zrow, zvec)

    return _tc_post(acc, den, bias, n)
